# baseline (speedup 1.0000x reference)
import functools

import jax
import jax.numpy as jnp
from jax import lax
from jax.experimental import pallas as pl
from jax.experimental.pallas import tpu as pltpu
from jax.experimental.pallas import tpu_sc as plsc

NUM_PATCHES = 576
PROJECTION_DIM = 768
BATCH = 64

NC = 2
NS = 16
NW = NC * NS
PW = NUM_PATCHES // NW
CHUNK = PW * PROJECTION_DIM
ROW = NUM_PATCHES * PROJECTION_DIM
TOTAL = BATCH * ROW
LANES = 16
NVEC = CHUNK // LANES


UNROLL = 8


def _sc_add(x_hbm, pos_hbm, out_hbm, pos_v, x_v, o_v, in_sem, out_sem):
    wid = lax.axis_index("s") * NC + lax.axis_index("c")
    base = wid * CHUNK
    pltpu.sync_copy(pos_hbm.at[pl.ds(base, CHUNK)], pos_v)

    pltpu.async_copy(x_hbm.at[pl.ds(base, CHUNK)], x_v.at[0], in_sem.at[0])

    def batch_body(b, _):
        cur = lax.rem(b, 2)
        nxt = lax.rem(b + 1, 2)
        off = b * ROW + base

        @pl.when(b + 1 < BATCH)
        def _start_next_in():
            noff = (b + 1) * ROW + base
            pltpu.async_copy(x_hbm.at[pl.ds(noff, CHUNK)], x_v.at[nxt],
                             in_sem.at[nxt])

        pltpu.make_async_copy(x_hbm.at[pl.ds(off, CHUNK)], x_v.at[cur],
                              in_sem.at[cur]).wait()

        @pl.when(b >= 2)
        def _drain_prev_out():
            poff = (b - 2) * ROW + base
            pltpu.make_async_copy(o_v.at[cur], out_hbm.at[pl.ds(poff, CHUNK)],
                                  out_sem.at[cur]).wait()

        def vec_body(j, _):
            for k in range(UNROLL):
                sl = pl.ds(j * (LANES * UNROLL) + k * LANES, LANES)
                o_v[cur, sl] = x_v[cur, sl] + pos_v[sl]
            return ()

        lax.fori_loop(0, NVEC // UNROLL, vec_body, ())
        pltpu.async_copy(o_v.at[cur], out_hbm.at[pl.ds(off, CHUNK)],
                         out_sem.at[cur])
        return ()

    lax.fori_loop(0, BATCH, batch_body, ())

    for b in (BATCH - 2, BATCH - 1):
        cur = b % 2
        pltpu.make_async_copy(o_v.at[cur],
                              out_hbm.at[pl.ds(b * ROW + base, CHUNK)],
                              out_sem.at[cur]).wait()


def kernel(x, pos_table):
    mesh = plsc.VectorSubcoreMesh(core_axis_name="c", subcore_axis_name="s")
    run = functools.partial(
        pl.kernel,
        out_type=jax.ShapeDtypeStruct((TOTAL,), jnp.float32),
        mesh=mesh,
        scratch_types=[
            pltpu.VMEM((CHUNK,), jnp.float32),
            pltpu.VMEM((2, CHUNK), jnp.float32),
            pltpu.VMEM((2, CHUNK), jnp.float32),
            pltpu.SemaphoreType.DMA((2,)),
            pltpu.SemaphoreType.DMA((2,)),
        ],
    )(_sc_add)
    out = run(x.reshape(-1), pos_table.reshape(-1))
    return out.reshape(x.shape)


# speedup vs baseline: 1.5937x; 1.5937x over previous
import functools

import jax
import jax.numpy as jnp
from jax import lax
from jax.experimental import pallas as pl
from jax.experimental.pallas import tpu as pltpu
from jax.experimental.pallas import tpu_sc as plsc

NUM_PATCHES = 576
PROJECTION_DIM = 768
BATCH = 64

NC = 2
NS = 16
NW = NC * NS
PW = NUM_PATCHES // NW
CHUNK = PW * PROJECTION_DIM
ROW = NUM_PATCHES * PROJECTION_DIM
TOTAL = BATCH * ROW
LANES = 16
NVEC = CHUNK // LANES


UNROLL = 8


def _sc_add(x_hbm, pos_hbm, out_hbm, pos_v, x_v, o_v, in_sem, out_sem):
    wid = lax.axis_index("s") * NC + lax.axis_index("c")
    base = wid * CHUNK
    pltpu.sync_copy(pos_hbm.at[pl.ds(base, CHUNK)], pos_v)

    pltpu.async_copy(x_hbm.at[pl.ds(base, CHUNK)], x_v.at[0], in_sem.at[0])

    def batch_body(b, _):
        cur = lax.rem(b, 2)
        nxt = lax.rem(b + 1, 2)
        off = b * ROW + base

        @pl.when(b + 1 < BATCH)
        def _start_next_in():
            noff = (b + 1) * ROW + base
            pltpu.async_copy(x_hbm.at[pl.ds(noff, CHUNK)], x_v.at[nxt],
                             in_sem.at[nxt])

        pltpu.make_async_copy(x_hbm.at[pl.ds(off, CHUNK)], x_v.at[cur],
                              in_sem.at[cur]).wait()

        @pl.when(b >= 2)
        def _drain_prev_out():
            poff = (b - 2) * ROW + base
            pltpu.make_async_copy(o_v.at[cur], out_hbm.at[pl.ds(poff, CHUNK)],
                                  out_sem.at[cur]).wait()

        @plsc.parallel_loop(0, NVEC, step=1, unroll=UNROLL)
        def _vec(i):
            sl = pl.ds(i * LANES, LANES)
            o_v[cur, sl] = x_v[cur, sl] + pos_v[sl]
        pltpu.async_copy(o_v.at[cur], out_hbm.at[pl.ds(off, CHUNK)],
                         out_sem.at[cur])
        return ()

    lax.fori_loop(0, BATCH, batch_body, ())

    for b in (BATCH - 2, BATCH - 1):
        cur = b % 2
        pltpu.make_async_copy(o_v.at[cur],
                              out_hbm.at[pl.ds(b * ROW + base, CHUNK)],
                              out_sem.at[cur]).wait()


def kernel(x, pos_table):
    mesh = plsc.VectorSubcoreMesh(core_axis_name="c", subcore_axis_name="s")
    run = functools.partial(
        pl.kernel,
        out_type=jax.ShapeDtypeStruct((TOTAL,), jnp.float32),
        mesh=mesh,
        scratch_types=[
            pltpu.VMEM((CHUNK,), jnp.float32),
            pltpu.VMEM((2, CHUNK), jnp.float32),
            pltpu.VMEM((2, CHUNK), jnp.float32),
            pltpu.SemaphoreType.DMA((2,)),
            pltpu.SemaphoreType.DMA((2,)),
        ],
    )(_sc_add)
    out = run(x.reshape(-1), pos_table.reshape(-1))
    return out.reshape(x.shape)


# speedup vs baseline: 1.6048x; 1.0069x over previous
import functools

import jax
import jax.numpy as jnp
from jax import lax
from jax.experimental import pallas as pl
from jax.experimental.pallas import tpu as pltpu
from jax.experimental.pallas import tpu_sc as plsc

NUM_PATCHES = 576
PROJECTION_DIM = 768
BATCH = 64

NC = 2
NS = 16
NW = NC * NS
PW = NUM_PATCHES // NW
CHUNK = PW * PROJECTION_DIM
ROW = NUM_PATCHES * PROJECTION_DIM
TOTAL = BATCH * ROW
LANES = 16
NVEC = CHUNK // LANES


UNROLL = 16


def _sc_add(x_hbm, pos_hbm, out_hbm, pos_v, x_v, o_v, in_sem, out_sem):
    wid = lax.axis_index("s") * NC + lax.axis_index("c")
    base = wid * CHUNK
    pltpu.sync_copy(pos_hbm.at[pl.ds(base, CHUNK)], pos_v)

    pltpu.async_copy(x_hbm.at[pl.ds(base, CHUNK)], x_v.at[0], in_sem.at[0])

    def batch_body(b, _):
        cur = lax.rem(b, 2)
        nxt = lax.rem(b + 1, 2)
        off = b * ROW + base

        @pl.when(b + 1 < BATCH)
        def _start_next_in():
            noff = (b + 1) * ROW + base
            pltpu.async_copy(x_hbm.at[pl.ds(noff, CHUNK)], x_v.at[nxt],
                             in_sem.at[nxt])

        pltpu.make_async_copy(x_hbm.at[pl.ds(off, CHUNK)], x_v.at[cur],
                              in_sem.at[cur]).wait()

        @pl.when(b >= 2)
        def _drain_prev_out():
            poff = (b - 2) * ROW + base
            pltpu.make_async_copy(o_v.at[cur], out_hbm.at[pl.ds(poff, CHUNK)],
                                  out_sem.at[cur]).wait()

        @plsc.parallel_loop(0, CHUNK, step=LANES, unroll=UNROLL)
        def _vec(i):
            sl = pl.ds(i, LANES)
            o_v[cur, sl] = x_v[cur, sl] + pos_v[sl]
        pltpu.async_copy(o_v.at[cur], out_hbm.at[pl.ds(off, CHUNK)],
                         out_sem.at[cur])
        return ()

    lax.fori_loop(0, BATCH, batch_body, ())

    for b in (BATCH - 2, BATCH - 1):
        cur = b % 2
        pltpu.make_async_copy(o_v.at[cur],
                              out_hbm.at[pl.ds(b * ROW + base, CHUNK)],
                              out_sem.at[cur]).wait()


def kernel(x, pos_table):
    mesh = plsc.VectorSubcoreMesh(core_axis_name="c", subcore_axis_name="s")
    run = functools.partial(
        pl.kernel,
        out_type=jax.ShapeDtypeStruct((TOTAL,), jnp.float32),
        mesh=mesh,
        scratch_types=[
            pltpu.VMEM((CHUNK,), jnp.float32),
            pltpu.VMEM((2, CHUNK), jnp.float32),
            pltpu.VMEM((2, CHUNK), jnp.float32),
            pltpu.SemaphoreType.DMA((2,)),
            pltpu.SemaphoreType.DMA((2,)),
        ],
    )(_sc_add)
    out = run(x.reshape(-1), pos_table.reshape(-1))
    return out.reshape(x.shape)


# speedup vs baseline: 2.5943x; 1.6166x over previous
import functools

import jax
import jax.numpy as jnp
from jax import lax
from jax.experimental import pallas as pl
from jax.experimental.pallas import tpu as pltpu
from jax.experimental.pallas import tpu_sc as plsc

NUM_PATCHES = 576
PROJECTION_DIM = 768
BATCH = 64

NC = 2
NS = 16
NW = NC * NS
LANES = 16

CP = 32
NCHUNK = NUM_PATCHES // CP
COLV = PROJECTION_DIM // LANES


def _sc_add(x_hbm, pos_hbm, out_hbm, p_v, x0_v, x1_v):
    wid = lax.axis_index("s") * NC + lax.axis_index("c")
    b0 = wid * 2

    def chunk_body(c, _):
        psl = pl.ds(c * CP, CP)
        pltpu.sync_copy(pos_hbm.at[psl], p_v)
        pltpu.sync_copy(x_hbm.at[b0, psl], x0_v)
        pltpu.sync_copy(x_hbm.at[b0 + 1, psl], x1_v)

        @plsc.parallel_loop(0, CP)
        def _row(r):
            for k in range(COLV):
                sl = pl.ds(k * LANES, LANES)
                pos_r = p_v[r, sl]
                x0_v[r, sl] += pos_r
                x1_v[r, sl] += pos_r

        pltpu.sync_copy(x0_v, out_hbm.at[b0, psl])
        pltpu.sync_copy(x1_v, out_hbm.at[b0 + 1, psl])
        return ()

    lax.fori_loop(0, NCHUNK, chunk_body, ())


def kernel(x, pos_table):
    mesh = plsc.VectorSubcoreMesh(core_axis_name="c", subcore_axis_name="s")
    run = functools.partial(
        pl.kernel,
        out_type=jax.ShapeDtypeStruct(x.shape, jnp.float32),
        mesh=mesh,
        scratch_types=[
            pltpu.VMEM((CP, PROJECTION_DIM), jnp.float32),
            pltpu.VMEM((CP, PROJECTION_DIM), jnp.float32),
            pltpu.VMEM((CP, PROJECTION_DIM), jnp.float32),
        ],
    )(_sc_add)
    return run(x, pos_table)


# speedup vs baseline: 5.4599x; 2.1046x over previous
import functools

import jax
import jax.numpy as jnp
from jax import lax
from jax.experimental import pallas as pl
from jax.experimental.pallas import tpu as pltpu
from jax.experimental.pallas import tpu_sc as plsc

NUM_PATCHES = 576
PROJECTION_DIM = 768
BATCH = 64

NC = 2
NS = 16
NW = NC * NS
LANES = 16
COLV = PROJECTION_DIM // LANES

NBG = 4
NTG = NW // NBG
BPW = BATCH // NBG
PPW = NUM_PATCHES // NTG
CP = 24
SPB = PPW // CP
NSTEP = BPW * SPB


def _sc_add(x_hbm, pos_hbm, out_hbm, p_v, x_v, o_v, in_sem, out_sem):
    wid = lax.axis_index("s") * NC + lax.axis_index("c")
    bg = lax.div(wid, NTG)
    tg = lax.rem(wid, NTG)
    b0 = bg * BPW
    p0 = tg * PPW

    pltpu.sync_copy(pos_hbm.at[pl.ds(p0, PPW)], p_v)

    def x_slice(s):
        b = b0 + lax.div(s, SPB)
        poff = p0 + lax.rem(s, SPB) * CP
        return x_hbm.at[b, pl.ds(poff, CP)]

    def out_slice(s):
        b = b0 + lax.div(s, SPB)
        poff = p0 + lax.rem(s, SPB) * CP
        return out_hbm.at[b, pl.ds(poff, CP)]

    pltpu.async_copy(x_slice(0), x_v.at[0], in_sem.at[0])

    def step_body(s, _):
        slot = lax.rem(s, 2)
        nslot = lax.rem(s + 1, 2)

        @pl.when(s + 1 < NSTEP)
        def _start_next_in():
            pltpu.async_copy(x_slice(s + 1), x_v.at[nslot], in_sem.at[nslot])

        pltpu.make_async_copy(x_slice(s), x_v.at[slot], in_sem.at[slot]).wait()

        @pl.when(s >= 2)
        def _drain_prev_out():
            pltpu.make_async_copy(o_v.at[slot], out_slice(s - 2),
                                  out_sem.at[slot]).wait()

        prow = lax.rem(s, SPB) * CP

        @plsc.parallel_loop(0, CP)
        def _row(r):
            for k in range(COLV):
                sl = pl.ds(k * LANES, LANES)
                o_v[slot, r, sl] = x_v[slot, r, sl] + p_v[prow + r, sl]

        pltpu.async_copy(o_v.at[slot], out_slice(s), out_sem.at[slot])
        return ()

    lax.fori_loop(0, NSTEP, step_body, ())

    for s in (NSTEP - 2, NSTEP - 1):
        slot = s % 2
        pltpu.make_async_copy(o_v.at[slot], out_slice(s),
                              out_sem.at[slot]).wait()


def kernel(x, pos_table):
    mesh = plsc.VectorSubcoreMesh(core_axis_name="c", subcore_axis_name="s")
    run = functools.partial(
        pl.kernel,
        out_type=jax.ShapeDtypeStruct(x.shape, jnp.float32),
        mesh=mesh,
        scratch_types=[
            pltpu.VMEM((PPW, PROJECTION_DIM), jnp.float32),
            pltpu.VMEM((2, CP, PROJECTION_DIM), jnp.float32),
            pltpu.VMEM((2, CP, PROJECTION_DIM), jnp.float32),
            pltpu.SemaphoreType.DMA((2,)),
            pltpu.SemaphoreType.DMA((2,)),
        ],
    )(_sc_add)
    return run(x, pos_table)
